# Initial kernel scaffold; baseline (speedup 1.0000x reference)
import functools

import jax
import jax.numpy as jnp
from jax import lax
from jax.experimental import pallas as pl
from jax.experimental.pallas import tpu as pltpu
from jax.experimental.pallas import tpu_sc as plsc

N_NODES = 100000
D = 128
NSEG = 256
NW = 32
CHUNK = 3136
BR = 112
NBLK = CHUNK // BR
NEG = -3.0e38


def _sc_body(x_hbm, ids_hbm, sums_hbm, maxs_hbm, cnts_hbm,
             ids_v, buf_v, sacc_v, macc_v, cacc_v, sem):
    wid = lax.axis_index("s") * 2 + lax.axis_index("c")
    row_lo = wid * CHUNK
    base = jnp.minimum(row_lo, N_NODES - CHUNK)

    zeros = jnp.zeros((16,), jnp.float32)
    negs = jnp.full((16,), NEG, jnp.float32)

    def init_row(i, _):
        for j in range(8):
            sacc_v[i, pl.ds(j * 16, 16)] = zeros
            macc_v[i, pl.ds(j * 16, 16)] = negs
        cacc_v[i, :] = zeros
        return 0
    lax.fori_loop(0, NSEG, init_row, 0)

    pltpu.sync_copy(ids_hbm.at[pl.ds(base, CHUNK)], ids_v)

    ones = jnp.ones((16,), jnp.float32)

    def block(bi, _):
        pltpu.sync_copy(x_hbm.at[pl.ds(base + bi * BR, BR), :], buf_v)

        def row(r, _):
            g = base + bi * BR + r

            @pl.when(g >= row_lo)
            def _():
                sid = ids_v[bi * BR + r]
                for j in range(8):
                    sl = pl.ds(j * 16, 16)
                    v = buf_v[r, sl]
                    sacc_v[sid, sl] = sacc_v[sid, sl] + v
                    macc_v[sid, sl] = jnp.maximum(macc_v[sid, sl], v)
                cacc_v[sid, :] = cacc_v[sid, :] + ones
            return 0
        lax.fori_loop(0, BR, row, 0)
        return 0
    lax.fori_loop(0, NBLK, block, 0)

    pltpu.sync_copy(sacc_v, sums_hbm.at[wid])
    pltpu.sync_copy(macc_v, maxs_hbm.at[wid])
    pltpu.sync_copy(cacc_v, cnts_hbm.at[wid])


def _sc_partials(x, ids):
    mesh = plsc.VectorSubcoreMesh(core_axis_name="c", subcore_axis_name="s")
    return pl.kernel(
        _sc_body,
        mesh=mesh,
        out_type=(
            jax.ShapeDtypeStruct((NW, NSEG, D), jnp.float32),
            jax.ShapeDtypeStruct((NW, NSEG, D), jnp.float32),
            jax.ShapeDtypeStruct((NW, NSEG, 16), jnp.float32),
        ),
        scratch_types=[
            pltpu.VMEM((CHUNK,), jnp.int32),
            pltpu.VMEM((BR, D), jnp.float32),
            pltpu.VMEM((NSEG, D), jnp.float32),
            pltpu.VMEM((NSEG, D), jnp.float32),
            pltpu.VMEM((NSEG, 16), jnp.float32),
            pltpu.SemaphoreType.DMA,
        ],
    )(x, ids)


def _tc_body(u_ref, sums_ref, maxs_ref, cnts_ref, w1_ref, b1_ref,
             w2_ref, b2_ref, y_ref):
    s = jnp.sum(sums_ref[...], axis=0)
    m = jnp.max(maxs_ref[...], axis=0)
    c16 = jnp.sum(cnts_ref[...], axis=(0, 2))[:, None]
    mean = s * 16.0 / jnp.maximum(c16, 16.0)
    m = jnp.where(c16 > 0.0, m, 0.0)
    u = u_ref[...]
    h = (jnp.dot(u, w1_ref[0:D, :], preferred_element_type=jnp.float32)
         + jnp.dot(mean, w1_ref[D:2 * D, :], preferred_element_type=jnp.float32)
         + jnp.dot(m, w1_ref[2 * D:3 * D, :], preferred_element_type=jnp.float32)
         + b1_ref[...][None, :])
    h = jnp.maximum(h, 0.0)
    y = jnp.dot(h, w2_ref[...], preferred_element_type=jnp.float32) + b2_ref[...][None, :]
    y_ref[...] = jnp.maximum(y, 0.0)


def _tc_combine(u, sums, maxs, cnts, W1, b1, W2, b2):
    return pl.pallas_call(
        _tc_body,
        out_shape=jax.ShapeDtypeStruct((NSEG, D), jnp.float32),
    )(u, sums, maxs, cnts, W1, b1, W2, b2)


@jax.jit
def kernel(x, edge_index, edge_attr, u, batch, W1, b1, W2, b2):
    ids = batch.astype(jnp.int32)
    sums, maxs, cnts = _sc_partials(x, ids)
    return _tc_combine(u, sums, maxs, cnts, W1, b1, W2, b2)


# speedup vs baseline: 7.3117x; 7.3117x over previous
import jax
import jax.numpy as jnp
from jax import lax
from jax.experimental import pallas as pl
from jax.experimental.pallas import tpu as pltpu
from jax.experimental.pallas import tpu_sc as plsc

N_NODES = 100000
D = 128
NSEG = 256
NW = 32
CHUNK = 3136
BR = 112
NBLK = CHUNK // BR
NRING = 4
NEG = -3.0e38


def _sc_body(x_hbm, ids_hbm, comb_hbm, cnts_hbm,
             ids_v, buf_v, cacc_v, reg_v, st_s, sem, isem):
    wid = lax.axis_index("s") * 2 + lax.axis_index("c")
    row_lo = wid * CHUNK
    base = jnp.minimum(row_lo, N_NODES - CHUNK)

    zeros = jnp.zeros((16,), jnp.float32)
    ones = jnp.ones((16,), jnp.float32)

    def init_row(i, _):
        cacc_v[i, :] = zeros
        return 0
    lax.fori_loop(0, NSEG, init_row, 0)

    st_s[0] = jnp.int32(-1)
    st_s[1] = jnp.int32(0)

    def flush():
        cur = st_s[0]
        cnt_run = st_s[1]

        @pl.when(cnt_run > 0)
        def _():
            pltpu.sync_copy(reg_v, comb_hbm.at[wid, cur])
            cacc_v[cur, :] = cacc_v[cur, :] + (zeros + cnt_run.astype(jnp.float32))

    def start(bi, b):
        pltpu.async_copy(x_hbm.at[pl.ds(base + bi * BR, BR), :],
                         buf_v.at[b], sem.at[b])
        pltpu.async_copy(ids_hbm.at[pl.ds(base + bi * BR, BR)],
                         ids_v.at[b], isem.at[b])

    for p in range(NRING - 1):
        start(p, p)

    def _process_block(bi, b4):
        def g_fn(gj, _):
            off = bi * BR + gj * 16
            r0 = gj * 16
            idvec = ids_v[b4, pl.ds(r0, 16)]
            sid0 = idvec[0]
            sidF = idvec[15]
            valid = (base + off) >= row_lo
            uniform = sid0 == sidF

            @pl.when(valid & uniform)
            def fast():
                cur = st_s[0]
                cnt_run = st_s[1]
                new = sid0 != cur

                @pl.when(new)
                def _():
                    flush()

                for j in range(8):
                    sl = pl.ds(j * 16, 16)
                    vs = [buf_v[b4, r0 + k, sl] for k in range(16)]
                    s = vs
                    while len(s) > 1:
                        s = [s[i] + s[i + 1] for i in range(0, len(s), 2)]
                    m = vs
                    while len(m) > 1:
                        m = [jnp.maximum(m[i], m[i + 1])
                             for i in range(0, len(m), 2)]
                    reg_v[0, sl] = jnp.where(new, s[0], reg_v[0, sl] + s[0])
                    reg_v[1, sl] = jnp.where(
                        new, m[0], jnp.maximum(reg_v[1, sl], m[0]))
                st_s[0] = sid0
                st_s[1] = jnp.where(new, jnp.int32(16), cnt_run + 16)

            @pl.when(valid & jnp.logical_not(uniform))
            def slow():
                for k in range(16):
                    sid = idvec[k]
                    new = sid != st_s[0]

                    @pl.when(new)
                    def _():
                        flush()

                    for j in range(8):
                        sl = pl.ds(j * 16, 16)
                        v = buf_v[b4, r0 + k, sl]
                        reg_v[0, sl] = jnp.where(new, v, reg_v[0, sl] + v)
                        reg_v[1, sl] = jnp.where(
                            new, v, jnp.maximum(reg_v[1, sl], v))
                    cnt_run = st_s[1]
                    st_s[0] = sid
                    st_s[1] = jnp.where(new, jnp.int32(1), cnt_run + 1)

            return 0

        lax.fori_loop(0, BR // 16, g_fn, 0)

    def ring_fn(bo, _):
        for b4 in range(NRING):
            bi = bo * NRING + b4
            pltpu.make_async_copy(x_hbm.at[pl.ds(base + bi * BR, BR), :],
                                  buf_v.at[b4], sem.at[b4]).wait()
            pltpu.make_async_copy(ids_hbm.at[pl.ds(base + bi * BR, BR)],
                                  ids_v.at[b4], isem.at[b4]).wait()

            @pl.when(bi + NRING - 1 < NBLK)
            def _():
                start(bi + NRING - 1, (b4 + NRING - 1) % NRING)

            _process_block(bi, b4)
        return 0

    lax.fori_loop(0, NBLK // NRING, ring_fn, 0)
    flush()

    pltpu.sync_copy(cacc_v, cnts_hbm.at[wid])


def _sc_partials(x, ids):
    mesh = plsc.VectorSubcoreMesh(core_axis_name="c", subcore_axis_name="s")
    return pl.kernel(
        _sc_body,
        mesh=mesh,
        out_type=(
            jax.ShapeDtypeStruct((NW, NSEG, 2, D), jnp.float32),
            jax.ShapeDtypeStruct((NW, NSEG, 16), jnp.float32),
        ),
        scratch_types=[
            pltpu.VMEM((NRING, BR), jnp.int32),
            pltpu.VMEM((NRING, BR, D), jnp.float32),
            pltpu.VMEM((NSEG, 16), jnp.float32),
            pltpu.VMEM((2, D), jnp.float32),
            pltpu.SMEM((2,), jnp.int32),
            pltpu.SemaphoreType.DMA((NRING,)),
            pltpu.SemaphoreType.DMA((NRING,)),
        ],
    )(x, ids)


def _tc_body(u_ref, comb_ref, cnts_ref, w1_ref, b1_ref,
             w2_ref, b2_ref, y_ref):
    cw = jnp.sum(cnts_ref[...], axis=2)
    valid = (cw > 0.0)[:, :, None]
    s = jnp.sum(jnp.where(valid, comb_ref[:, :, 0, :], 0.0), axis=0)
    m = jnp.max(jnp.where(valid, comb_ref[:, :, 1, :], NEG), axis=0)
    c16 = jnp.sum(cw, axis=0)[:, None]
    mean = s * 16.0 / jnp.maximum(c16, 16.0)
    m = jnp.where(c16 > 0.0, m, 0.0)
    u = u_ref[...]
    h = (jnp.dot(u, w1_ref[0:D, :], preferred_element_type=jnp.float32)
         + jnp.dot(mean, w1_ref[D:2 * D, :], preferred_element_type=jnp.float32)
         + jnp.dot(m, w1_ref[2 * D:3 * D, :], preferred_element_type=jnp.float32)
         + b1_ref[...][None, :])
    h = jnp.maximum(h, 0.0)
    y = jnp.dot(h, w2_ref[...], preferred_element_type=jnp.float32) + b2_ref[...][None, :]
    y_ref[...] = jnp.maximum(y, 0.0)


def _tc_combine(u, comb, cnts, W1, b1, W2, b2):
    return pl.pallas_call(
        _tc_body,
        out_shape=jax.ShapeDtypeStruct((NSEG, D), jnp.float32),
    )(u, comb, cnts, W1, b1, W2, b2)


@jax.jit
def kernel(x, edge_index, edge_attr, u, batch, W1, b1, W2, b2):
    ids = batch.astype(jnp.int32)
    comb, cnts = _sc_partials(x, ids)
    return _tc_combine(u, comb, cnts, W1, b1, W2, b2)


# speedup vs baseline: 7.3932x; 1.0112x over previous
import jax
import jax.numpy as jnp
from jax import lax
from jax.experimental import pallas as pl
from jax.experimental.pallas import tpu as pltpu
from jax.experimental.pallas import tpu_sc as plsc

N_NODES = 100000
D = 128
NSEG = 256
NW = 32
CHUNK = 3136
BR = 112
NBLK = CHUNK // BR
NRING = 4
NEG = -3.0e38


def _sc_body(x_hbm, ids_hbm, sums_hbm, maxs_hbm, cnts_hbm,
             ids_v, buf_v, sacc_v, macc_v, cacc_v, reg_v, st_s, sem, isem):
    wid = lax.axis_index("s") * 2 + lax.axis_index("c")
    row_lo = wid * CHUNK
    base = jnp.minimum(row_lo, N_NODES - CHUNK)

    zeros = jnp.zeros((16,), jnp.float32)
    negs = jnp.full((16,), NEG, jnp.float32)
    lanes = jnp.arange(16, dtype=jnp.int32)

    for i in range(16):
        cacc_v[i, :] = zeros

    st_s[0] = jnp.int32(-1)
    st_s[1] = jnp.int32(0)

    def flush():
        cur = st_s[0]
        cnt_run = st_s[1]

        @pl.when(cnt_run > 0)
        def _():
            for j in range(8):
                sl = pl.ds(j * 16, 16)
                sacc_v[cur, sl] = reg_v[0, sl] + zeros
                macc_v[cur, sl] = jnp.maximum(reg_v[1, sl], negs)
            hi = cur // 16
            cf = zeros + cnt_run.astype(jnp.float32)
            cacc_v[hi, :] = jnp.where(lanes == (cur % 16), cf, cacc_v[hi, :])

    def start(bi, b):
        pltpu.async_copy(x_hbm.at[pl.ds(base + bi * BR, BR), :],
                         buf_v.at[b], sem.at[b])
        pltpu.async_copy(ids_hbm.at[pl.ds(base + bi * BR, BR)],
                         ids_v.at[b, pl.ds(0, BR)], isem.at[b])

    for p in range(NRING - 1):
        start(p, p)

    def _process_block(bi, b4):
        def g_fn(gj, _):
            off = bi * BR + gj * 16
            r0 = gj * 16
            idvec = ids_v[b4, pl.ds(r0, 16)]
            sid0 = idvec[0]
            sidF = idvec[15]
            valid = (base + off) >= row_lo
            uniform = sid0 == sidF

            @pl.when(valid & uniform)
            def fast():
                cur = st_s[0]
                cnt_run = st_s[1]
                new = sid0 != cur

                @pl.when(new)
                def _():
                    flush()

                for j in range(8):
                    sl = pl.ds(j * 16, 16)
                    vs = [buf_v[b4, r0 + k, sl] for k in range(16)]
                    s = vs
                    while len(s) > 1:
                        s = [s[i] + s[i + 1] for i in range(0, len(s), 2)]
                    m = vs
                    while len(m) > 1:
                        m = [jnp.maximum(m[i], m[i + 1])
                             for i in range(0, len(m), 2)]
                    reg_v[0, sl] = jnp.where(new, s[0], reg_v[0, sl] + s[0])
                    reg_v[1, sl] = jnp.where(
                        new, m[0], jnp.maximum(reg_v[1, sl], m[0]))
                st_s[0] = sid0
                st_s[1] = jnp.where(new, jnp.int32(16), cnt_run + 16)

            @pl.when(valid & jnp.logical_not(uniform))
            def slow():
                def row_fn(k, _):
                    sid = ids_v[b4, pl.ds(r0 + k, 16)][0]
                    new = sid != st_s[0]

                    @pl.when(new)
                    def _():
                        flush()

                    for j in range(8):
                        sl = pl.ds(j * 16, 16)
                        v = buf_v[b4, r0 + k, sl]
                        reg_v[0, sl] = jnp.where(new, v, reg_v[0, sl] + v)
                        reg_v[1, sl] = jnp.where(
                            new, v, jnp.maximum(reg_v[1, sl], v))
                    cnt_run = st_s[1]
                    st_s[0] = sid
                    st_s[1] = jnp.where(new, jnp.int32(1), cnt_run + 1)
                    return 0

                lax.fori_loop(0, 16, row_fn, 0)

            return 0

        lax.fori_loop(0, BR // 16, g_fn, 0)

    def ring_fn(bi, _):
        b4 = bi & (NRING - 1)
        pltpu.make_async_copy(x_hbm.at[pl.ds(base + bi * BR, BR), :],
                              buf_v.at[b4], sem.at[b4]).wait()
        pltpu.make_async_copy(ids_hbm.at[pl.ds(base + bi * BR, BR)],
                              ids_v.at[b4, pl.ds(0, BR)], isem.at[b4]).wait()

        @pl.when(bi + NRING - 1 < NBLK)
        def _():
            start(bi + NRING - 1, (bi + NRING - 1) & (NRING - 1))

        _process_block(bi, b4)
        return 0

    lax.fori_loop(0, NBLK, ring_fn, 0)
    flush()

    pltpu.sync_copy(sacc_v, sums_hbm.at[wid])
    pltpu.sync_copy(macc_v, maxs_hbm.at[wid])
    pltpu.sync_copy(cacc_v, cnts_hbm.at[wid])


def _sc_partials(x, ids):
    mesh = plsc.VectorSubcoreMesh(core_axis_name="c", subcore_axis_name="s")
    return pl.kernel(
        _sc_body,
        mesh=mesh,
        out_type=(
            jax.ShapeDtypeStruct((NW, NSEG, D), jnp.float32),
            jax.ShapeDtypeStruct((NW, NSEG, D), jnp.float32),
            jax.ShapeDtypeStruct((NW, 16, 16), jnp.float32),
        ),
        scratch_types=[
            pltpu.VMEM((NRING, 128), jnp.int32),
            pltpu.VMEM((NRING, BR, D), jnp.float32),
            pltpu.VMEM((NSEG, D), jnp.float32),
            pltpu.VMEM((NSEG, D), jnp.float32),
            pltpu.VMEM((16, 16), jnp.float32),
            pltpu.VMEM((2, D), jnp.float32),
            pltpu.SMEM((2,), jnp.int32),
            pltpu.SemaphoreType.DMA((NRING,)),
            pltpu.SemaphoreType.DMA((NRING,)),
        ],
    )(x, ids)


def _tc_body(u_ref, sums_ref, maxs_ref, cnts_ref, w1_ref, b1_ref,
             w2_ref, b2_ref, y_ref):
    cw = cnts_ref[...][:, :, None]
    valid = cw > 0.0
    s = jnp.sum(jnp.where(valid, sums_ref[...], 0.0), axis=0)
    m = jnp.max(jnp.where(valid, maxs_ref[...], NEG), axis=0)
    c = jnp.sum(cw, axis=0)
    mean = s / jnp.maximum(c, 1.0)
    m = jnp.where(c > 0.0, m, 0.0)
    u = u_ref[...]
    h = (jnp.dot(u, w1_ref[0:D, :], preferred_element_type=jnp.float32)
         + jnp.dot(mean, w1_ref[D:2 * D, :], preferred_element_type=jnp.float32)
         + jnp.dot(m, w1_ref[2 * D:3 * D, :], preferred_element_type=jnp.float32)
         + b1_ref[...][None, :])
    h = jnp.maximum(h, 0.0)
    y = jnp.dot(h, w2_ref[...], preferred_element_type=jnp.float32) + b2_ref[...][None, :]
    y_ref[...] = jnp.maximum(y, 0.0)


def _tc_combine(u, sums, maxs, cnts, W1, b1, W2, b2):
    return pl.pallas_call(
        _tc_body,
        out_shape=jax.ShapeDtypeStruct((NSEG, D), jnp.float32),
    )(u, sums, maxs, cnts, W1, b1, W2, b2)


@jax.jit
def kernel(x, edge_index, edge_attr, u, batch, W1, b1, W2, b2):
    ids = batch.astype(jnp.int32)
    sums, maxs, cnts = _sc_partials(x, ids)
    cnts = cnts.reshape(NW, NSEG)
    return _tc_combine(u, sums, maxs, cnts, W1, b1, W2, b2)


# speedup vs baseline: 7.4199x; 1.0036x over previous
import jax
import jax.numpy as jnp
from jax import lax
from jax.experimental import pallas as pl
from jax.experimental.pallas import tpu as pltpu
from jax.experimental.pallas import tpu_sc as plsc

N_NODES = 100000
D = 128
NSEG = 256
NW = 32
CHUNK = 3136
BR = 224
NBLK = CHUNK // BR
NRING = 2
NEG = -3.0e38


def _sc_body(x_hbm, ids_hbm, sums_hbm, maxs_hbm, cnts_hbm,
             ids_v, buf_v, sacc_v, macc_v, cacc_v, reg_v, st_s, sem, isem):
    wid = lax.axis_index("s") * 2 + lax.axis_index("c")
    row_lo = wid * CHUNK
    base = jnp.minimum(row_lo, N_NODES - CHUNK)

    zeros = jnp.zeros((16,), jnp.float32)
    negs = jnp.full((16,), NEG, jnp.float32)
    lanes = jnp.arange(16, dtype=jnp.int32)

    for i in range(16):
        cacc_v[i, :] = zeros

    st_s[0] = jnp.int32(-1)
    st_s[1] = jnp.int32(0)

    def flush():
        cur = st_s[0]
        cnt_run = st_s[1]

        @pl.when(cnt_run > 0)
        def _():
            for j in range(8):
                sl = pl.ds(j * 16, 16)
                sacc_v[cur, sl] = reg_v[0, sl] + zeros
                macc_v[cur, sl] = jnp.maximum(reg_v[1, sl], negs)
            hi = cur // 16
            cf = zeros + cnt_run.astype(jnp.float32)
            cacc_v[hi, :] = jnp.where(lanes == (cur % 16), cf, cacc_v[hi, :])

    def start(bi, b):
        pltpu.async_copy(x_hbm.at[pl.ds(base + bi * BR, BR), :],
                         buf_v.at[b], sem.at[b])
        pltpu.async_copy(ids_hbm.at[pl.ds(base + bi * BR, BR)],
                         ids_v.at[pl.ds(b * 256, BR)], isem.at[b])

    for p in range(NRING - 1):
        start(p, p)

    def _process_block(bi, b4):
        def g_fn(gj, _):
            off = bi * BR + gj * 16
            r0 = gj * 16
            idvec = ids_v[pl.ds(b4 * 256 + r0, 16)]
            sid0 = idvec[0]
            sidF = idvec[15]
            valid = (base + off) >= row_lo
            uniform = sid0 == sidF

            @pl.when(valid & uniform)
            def fast():
                cur = st_s[0]
                cnt_run = st_s[1]
                new = sid0 != cur

                @pl.when(new)
                def _():
                    flush()

                for j in range(8):
                    sl = pl.ds(j * 16, 16)
                    vs = [buf_v[b4, r0 + k, sl] for k in range(16)]
                    s = vs
                    while len(s) > 1:
                        s = [s[i] + s[i + 1] for i in range(0, len(s), 2)]
                    m = vs
                    while len(m) > 1:
                        m = [jnp.maximum(m[i], m[i + 1])
                             for i in range(0, len(m), 2)]
                    reg_v[0, sl] = jnp.where(new, s[0], reg_v[0, sl] + s[0])
                    reg_v[1, sl] = jnp.where(
                        new, m[0], jnp.maximum(reg_v[1, sl], m[0]))
                st_s[0] = sid0
                st_s[1] = jnp.where(new, jnp.int32(16), cnt_run + 16)

            @pl.when(valid & jnp.logical_not(uniform))
            def slow():
                def row_fn(k, _):
                    sid = ids_v[pl.ds(b4 * 256 + r0 + k, 16)][0]
                    new = sid != st_s[0]

                    @pl.when(new)
                    def _():
                        flush()

                    for j in range(8):
                        sl = pl.ds(j * 16, 16)
                        v = buf_v[b4, r0 + k, sl]
                        reg_v[0, sl] = jnp.where(new, v, reg_v[0, sl] + v)
                        reg_v[1, sl] = jnp.where(
                            new, v, jnp.maximum(reg_v[1, sl], v))
                    cnt_run = st_s[1]
                    st_s[0] = sid
                    st_s[1] = jnp.where(new, jnp.int32(1), cnt_run + 1)
                    return 0

                lax.fori_loop(0, 16, row_fn, 0)

            return 0

        lax.fori_loop(0, BR // 16, g_fn, 0)

    def ring_fn(bi, _):
        b4 = bi & (NRING - 1)
        pltpu.make_async_copy(x_hbm.at[pl.ds(base + bi * BR, BR), :],
                              buf_v.at[b4], sem.at[b4]).wait()
        pltpu.make_async_copy(ids_hbm.at[pl.ds(base + bi * BR, BR)],
                              ids_v.at[pl.ds(b4 * 256, BR)], isem.at[b4]).wait()

        @pl.when(bi + NRING - 1 < NBLK)
        def _():
            start(bi + NRING - 1, (bi + NRING - 1) & (NRING - 1))

        _process_block(bi, b4)
        return 0

    lax.fori_loop(0, NBLK, ring_fn, 0)
    flush()

    pltpu.sync_copy(sacc_v, sums_hbm.at[wid])
    pltpu.sync_copy(macc_v, maxs_hbm.at[wid])
    pltpu.sync_copy(cacc_v, cnts_hbm.at[wid])


def _sc_partials(x, ids):
    mesh = plsc.VectorSubcoreMesh(core_axis_name="c", subcore_axis_name="s")
    return pl.kernel(
        _sc_body,
        mesh=mesh,
        out_type=(
            jax.ShapeDtypeStruct((NW, NSEG, D), jnp.float32),
            jax.ShapeDtypeStruct((NW, NSEG, D), jnp.float32),
            jax.ShapeDtypeStruct((NW, 16, 16), jnp.float32),
        ),
        scratch_types=[
            pltpu.VMEM((NRING * 256,), jnp.int32),
            pltpu.VMEM((NRING, BR, D), jnp.float32),
            pltpu.VMEM((NSEG, D), jnp.float32),
            pltpu.VMEM((NSEG, D), jnp.float32),
            pltpu.VMEM((16, 16), jnp.float32),
            pltpu.VMEM((2, D), jnp.float32),
            pltpu.SMEM((2,), jnp.int32),
            pltpu.SemaphoreType.DMA((NRING,)),
            pltpu.SemaphoreType.DMA((NRING,)),
        ],
    )(x, ids)


def _tc_body(u_ref, sums_ref, maxs_ref, cnts_ref, w1_ref, b1_ref,
             w2_ref, b2_ref, y_ref):
    cw = cnts_ref[...][:, :, None]
    valid = cw > 0.0
    s = jnp.sum(jnp.where(valid, sums_ref[...], 0.0), axis=0)
    m = jnp.max(jnp.where(valid, maxs_ref[...], NEG), axis=0)
    c = jnp.sum(cw, axis=0)
    mean = s / jnp.maximum(c, 1.0)
    m = jnp.where(c > 0.0, m, 0.0)
    u = u_ref[...]
    h = (jnp.dot(u, w1_ref[0:D, :], preferred_element_type=jnp.float32)
         + jnp.dot(mean, w1_ref[D:2 * D, :], preferred_element_type=jnp.float32)
         + jnp.dot(m, w1_ref[2 * D:3 * D, :], preferred_element_type=jnp.float32)
         + b1_ref[...][None, :])
    h = jnp.maximum(h, 0.0)
    y = jnp.dot(h, w2_ref[...], preferred_element_type=jnp.float32) + b2_ref[...][None, :]
    y_ref[...] = jnp.maximum(y, 0.0)


def _tc_combine(u, sums, maxs, cnts, W1, b1, W2, b2):
    return pl.pallas_call(
        _tc_body,
        out_shape=jax.ShapeDtypeStruct((NSEG, D), jnp.float32),
    )(u, sums, maxs, cnts, W1, b1, W2, b2)


@jax.jit
def kernel(x, edge_index, edge_attr, u, batch, W1, b1, W2, b2):
    ids = batch.astype(jnp.int32)
    sums, maxs, cnts = _sc_partials(x, ids)
    cnts = cnts.reshape(NW, NSEG)
    return _tc_combine(u, sums, maxs, cnts, W1, b1, W2, b2)
